# Initial kernel scaffold; baseline (speedup 1.0000x reference)
import jax
import jax.numpy as jnp
from jax.experimental import pallas as pl


def kernel(x, edge_index, W1, b1, W2, b2):
    raise NotImplementedError("write your pallas kernel here")


# speedup vs baseline: 9.0974x; 9.0974x over previous
import functools

import jax
import jax.numpy as jnp
from jax import lax
from jax.experimental import pallas as pl
from jax.experimental.pallas import tpu as pltpu
from jax.experimental.pallas import tpu_sc as plsc

N = 10000
E = 320000
D = 128
L = 128
NCHUNK = E // L
NC, NS = 2, 16
NW = NC * NS
CPW = NCHUNK // NW
CREM = NCHUNK % NW
RPT = 624
RLAST = N - (NS - 1) * RPT - RPT
RB = 200
GRID = N // RB

_mesh = plsc.VectorSubcoreMesh(core_axis_name="c", subcore_axis_name="s")
_sc_params = pltpu.CompilerParams(needs_layout_passes=False)


@functools.partial(
    pl.kernel,
    out_type=jax.ShapeDtypeStruct((NC * N,), jnp.float32),
    mesh=_mesh,
    compiler_params=_sc_params,
    scratch_types=[
        pltpu.VMEM((L,), jnp.int32),
        pltpu.VMEM((RPT + 16,), jnp.float32),
        pltpu.VMEM((L,), jnp.float32),
        pltpu.VMEM_SHARED((N,), jnp.float32),
    ],
)
def _deg(dst_hbm, out_hbm, didx, zeros_v, ones_v, acc):
    cid = lax.axis_index("c")
    sid = lax.axis_index("s")
    w = sid * NC + cid
    r0 = pl.multiple_of(sid * RPT, 8)

    def zero_body(i, carry):
        zeros_v[pl.ds(i * 16, 16)] = jnp.zeros((16,), jnp.float32)
        return carry

    lax.fori_loop(0, (RPT + 16) // 16, zero_body, 0)

    def ones_body(i, carry):
        ones_v[pl.ds(i * 16, 16)] = jnp.ones((16,), jnp.float32)
        return carry

    lax.fori_loop(0, L // 16, ones_body, 0)

    pltpu.sync_copy(zeros_v.at[pl.ds(0, RPT)], acc.at[pl.ds(r0, RPT)])

    @pl.when(sid == NS - 1)
    def _init_tail():
        pltpu.sync_copy(zeros_v.at[pl.ds(RPT, RLAST)],
                        acc.at[pl.ds(NS * RPT, RLAST)])

    plsc.subcore_barrier()

    lo = w * CPW + jnp.minimum(w, CREM)
    n = CPW + (w < CREM).astype(jnp.int32)

    def chunk_body(j, carry):
        pltpu.sync_copy(dst_hbm.at[pl.ds(pl.multiple_of(j * L, L), L)], didx)
        pltpu.sync_copy(ones_v, acc.at[didx], add=True)
        return carry

    lax.fori_loop(lo, lo + n, chunk_body, 0)
    plsc.subcore_barrier()

    o0 = pl.multiple_of(cid * N + r0, 8)
    pltpu.sync_copy(acc.at[pl.ds(r0, RPT)], zeros_v.at[pl.ds(0, RPT)])
    pltpu.sync_copy(zeros_v.at[pl.ds(0, RPT)], out_hbm.at[pl.ds(o0, RPT)])

    @pl.when(sid == NS - 1)
    def _read_tail():
        pltpu.sync_copy(acc.at[pl.ds(NS * RPT, RLAST)],
                        zeros_v.at[pl.ds(RPT, RLAST)])
        pltpu.sync_copy(
            zeros_v.at[pl.ds(RPT, RLAST)],
            out_hbm.at[pl.ds(pl.multiple_of(cid * N + NS * RPT, 8), RLAST)])


NBUF = 2
assert CPW % NBUF == 0


@functools.partial(
    pl.kernel,
    out_type=jax.ShapeDtypeStruct((NC * N, D), jnp.float32),
    mesh=_mesh,
    compiler_params=_sc_params,
    scratch_types=[
        pltpu.VMEM((L,), jnp.int32),
        pltpu.VMEM((L,), jnp.int32),
        pltpu.VMEM((L,), jnp.int32),
        pltpu.VMEM((L,), jnp.int32),
        pltpu.VMEM((L,), jnp.int32),
        pltpu.VMEM((L, D), jnp.float32),
        pltpu.VMEM((L, D), jnp.float32),
        pltpu.VMEM((N + 8,), jnp.int32),
        pltpu.VMEM_SHARED((N + 8, D), jnp.float32),
        pltpu.SemaphoreType.DMA,
        pltpu.SemaphoreType.DMA,
        pltpu.SemaphoreType.DMA,
    ],
)
def _agg(g_hbm, src_hbm, dst_hbm, out_hbm,
         sidx, didx0, didx1, wbuf, tbuf, rows0, rows1, probe, acc,
         gsem, ssem0, ssem1):
    didx = [didx0, didx1]
    rows = [rows0, rows1]
    ssem = [ssem0, ssem1]
    cid = lax.axis_index("c")
    sid = lax.axis_index("s")
    w = sid * NC + cid
    r0 = pl.multiple_of(sid * RPT, 8)
    DUM = N

    def _peel(R, W_sc, W_rem):
        for j in range(L // 16):
            idx_j = R[pl.ds(j * 16, 16)]
            pvec = lax.iota(jnp.int32, 16) + (j * 16)
            plsc.store_scatter(probe, [idx_j], pvec)
        rem = jnp.int32(0)
        for j in range(L // 16):
            idx_j = R[pl.ds(j * 16, 16)]
            pvec = lax.iota(jnp.int32, 16) + (j * 16)
            q_j = plsc.load_gather(probe, [idx_j])
            win = (q_j == pvec) & (idx_j != DUM)
            W_sc[pl.ds(j * 16, 16)] = jnp.where(win, idx_j, DUM)
            rem_j = jnp.where(win, DUM, idx_j)
            W_rem[pl.ds(j * 16, 16)] = rem_j
            rem = rem + plsc.all_reduce_population_count(rem_j != DUM)[0]
        return rem

    def _scatter_chunk_sync(didx_ref, rows_ref):
        rem = _peel(didx_ref, didx_ref, wbuf)
        pltpu.sync_copy(rows_ref, acc.at[didx_ref], add=True)

        def wcond(rem):
            return rem > 0

        def wbody(rem):
            rem2 = _peel(wbuf, tbuf, wbuf)
            pltpu.sync_copy(rows_ref, acc.at[tbuf], add=True)
            return rem2

        lax.while_loop(wcond, wbody, rem)

    def _init_rows(off, nr, buf):
        pltpu.sync_copy(g_hbm.at[pl.ds(off, nr)], buf.at[pl.ds(0, nr)])
        pltpu.sync_copy(buf.at[pl.ds(0, nr)], acc.at[pl.ds(off, nr)])

    for k in range(4):
        _init_rows(r0 + k * L, L, rows[k % NBUF])
    _init_rows(r0 + 4 * L, RPT - 4 * L, rows0)

    @pl.when(sid == NS - 1)
    def _init_tail():
        _init_rows(pl.multiple_of(NS * RPT, 8), RLAST, rows1)

    pl.delay(100_000)
    plsc.subcore_barrier()

    lo = w * CPW

    def group_body(gi, carry):
        for b in range(NBUF):
            e0 = pl.multiple_of((lo + gi * NBUF + b) * L, L)

            @pl.when(gi > 0)
            def _drain():
                pltpu.make_async_copy(rows[b], acc.at[didx[b]],
                                      ssem[b]).wait()

            pltpu.sync_copy(dst_hbm.at[pl.ds(e0, L)], didx[b])
            pltpu.sync_copy(src_hbm.at[pl.ds(e0, L)], sidx)
            pltpu.async_copy(g_hbm.at[sidx], rows[b], gsem).wait()

            rem = _peel(didx[b], didx[b], wbuf)

            def wbody(r, rows_b=rows[b]):
                rem2 = _peel(wbuf, tbuf, wbuf)
                pltpu.sync_copy(rows_b, acc.at[tbuf], add=True)
                return rem2

            lax.while_loop(lambda r: r > 0, wbody, rem)
            pltpu.async_copy(rows[b], acc.at[didx[b]], ssem[b], add=True)
        return carry

    lax.fori_loop(0, CPW // NBUF, group_body, 0)
    for b in range(NBUF):
        pltpu.make_async_copy(rows[b], acc.at[didx[b]], ssem[b]).wait()

    @pl.when(w < CREM)
    def _extra():
        e0 = pl.multiple_of((NW * CPW + w) * L, L)
        pltpu.sync_copy(dst_hbm.at[pl.ds(e0, L)], didx0)
        pltpu.sync_copy(src_hbm.at[pl.ds(e0, L)], sidx)
        pltpu.async_copy(g_hbm.at[sidx], rows0, gsem).wait()
        _scatter_chunk_sync(didx0, rows0)

    pl.delay(100_000)
    plsc.subcore_barrier()

    def _read_rows(aoff, ooff, nr, buf):
        pltpu.sync_copy(acc.at[pl.ds(aoff, nr)], buf.at[pl.ds(0, nr)])
        pltpu.sync_copy(buf.at[pl.ds(0, nr)], out_hbm.at[pl.ds(ooff, nr)])

    o0 = pl.multiple_of(cid * N + r0, 8)
    for k in range(4):
        _read_rows(r0 + k * L, o0 + k * L, L, rows[k % NBUF])
    _read_rows(r0 + 4 * L, o0 + 4 * L, RPT - 4 * L, rows0)

    @pl.when(sid == NS - 1)
    def _read_tail():
        _read_rows(pl.multiple_of(NS * RPT, 8),
                   pl.multiple_of(cid * N + NS * RPT, 8), RLAST, rows1)


def _dinv_of(degT_blk):
    deg = jnp.sum(degT_blk, axis=1) + 1.0
    return lax.rsqrt(deg)


def _mm1_body(x_ref, w_ref, degT_ref, o_ref):
    dinv = _dinv_of(degT_ref[...])
    h = jnp.dot(x_ref[...], w_ref[...], preferred_element_type=jnp.float32)
    o_ref[...] = h * dinv[:, None]


def _mm2_body(pa_ref, pb_ref, g1_ref, degT_ref, w_ref, b_ref, o_ref):
    dinv = _dinv_of(degT_ref[...])
    agg = pa_ref[...] + pb_ref[...] - g1_ref[...]
    t = jnp.maximum(agg * dinv[:, None] + b_ref[...], 0.0)
    h = jnp.dot(t, w_ref[...], preferred_element_type=jnp.float32)
    o_ref[...] = h * dinv[:, None]


def _fin_body(pa_ref, pb_ref, g2_ref, degT_ref, b_ref, o_ref):
    dinv = _dinv_of(degT_ref[...])
    agg = pa_ref[...] + pb_ref[...] - g2_ref[...]
    o = agg * dinv[:, None] + b_ref[...]
    nrm = jnp.sqrt(jnp.sum(o * o, axis=1, keepdims=True))
    o = o / jnp.maximum(nrm, 1e-12)
    o_ref[...] = jax.nn.sigmoid(o)


_row_spec = pl.BlockSpec((RB, D), lambda i: (i, 0))
_rowB_spec = pl.BlockSpec((RB, D), lambda i: (i + GRID, 0))
_degT_spec = pl.BlockSpec((RB, NC), lambda i: (i, 0))
_w_spec = pl.BlockSpec((D, D), lambda i: (0, 0))
_b_spec = pl.BlockSpec((1, D), lambda i: (0, 0))
_out_rows = jax.ShapeDtypeStruct((N, D), jnp.float32)

_mm1 = pl.pallas_call(
    _mm1_body, grid=(GRID,),
    in_specs=[_row_spec, _w_spec, _degT_spec],
    out_specs=_row_spec, out_shape=_out_rows)

_mm2 = pl.pallas_call(
    _mm2_body, grid=(GRID,),
    in_specs=[_row_spec, _rowB_spec, _row_spec, _degT_spec, _w_spec, _b_spec],
    out_specs=_row_spec, out_shape=_out_rows)

_fin = pl.pallas_call(
    _fin_body, grid=(GRID,),
    in_specs=[_row_spec, _rowB_spec, _row_spec, _degT_spec, _b_spec],
    out_specs=_row_spec, out_shape=_out_rows)


def kernel(x, edge_index, W1, b1, W2, b2):
    src = edge_index[0].astype(jnp.int32)
    dst = edge_index[1].astype(jnp.int32)
    degT = _deg(dst).reshape(NC, N).T
    b1r = b1.reshape(1, D)
    b2r = b2.reshape(1, D)

    g1 = _mm1(x, W1, degT)
    p1 = _agg(g1, src, dst)
    g2 = _mm2(p1[:N], p1[N:], g1, degT, W2, b1r)
    p2 = _agg(g2, src, dst)
    return _fin(p2[:N], p2[N:], g2, degT, b2r)
